# Initial kernel scaffold; baseline (speedup 1.0000x reference)
import jax
import jax.numpy as jnp
from jax.experimental import pallas as pl


def kernel(table, x):
    raise NotImplementedError("write your pallas kernel here")


# speedup vs baseline: 1.1016x; 1.1016x over previous
import functools

import jax
import jax.numpy as jnp
from jax import lax
from jax.experimental import pallas as pl
from jax.experimental.pallas import tpu as pltpu
from jax.experimental.pallas import tpu_sc as plsc

_NUM_CORES = 2
_NUM_SUBCORES = 16
_NUM_WORKERS = _NUM_CORES * _NUM_SUBCORES


@functools.partial(jax.jit, static_argnums=(2, 3, 4))
def _sc_gather(table, idx, B, D, C):
    b_per_w = B // _NUM_WORKERS
    n_chunks = b_per_w // C
    mesh = plsc.VectorSubcoreMesh(core_axis_name="c", subcore_axis_name="s")

    @functools.partial(
        pl.kernel,
        mesh=mesh,
        out_type=jax.ShapeDtypeStruct((B, D), jnp.float32),
        scratch_types=[
            pltpu.VMEM((C,), jnp.int32),
            pltpu.VMEM((C, D), jnp.float32),
            pltpu.SemaphoreType.DMA,
        ],
        compiler_params=pltpu.CompilerParams(use_tc_tiling_on_sc=False),
    )
    def k(table_hbm, idx_hbm, out_hbm, idx_v, rows_v, sem):
        wid = lax.axis_index("s") * _NUM_CORES + lax.axis_index("c")
        base = wid * b_per_w

        def body(i, carry):
            off = base + i * C
            pltpu.sync_copy(idx_hbm.at[pl.ds(off, C)], idx_v)
            pltpu.async_copy(table_hbm.at[idx_v], rows_v, sem).wait()
            pltpu.sync_copy(rows_v, out_hbm.at[pl.ds(off, C)])
            return carry

        lax.fori_loop(0, n_chunks, body, 0)

    return k(table, idx)


def kernel(table, x):
    batch, hist = x.shape
    D = table.shape[1]
    B = batch * hist
    idx = x.reshape(-1).astype(jnp.int32)
    out = _sc_gather(table, idx, B, D, 1600)
    return out.reshape(batch, hist, D)


# speedup vs baseline: 1.1133x; 1.0106x over previous
import functools

import jax
import jax.numpy as jnp
from jax import lax
from jax.experimental import pallas as pl
from jax.experimental.pallas import tpu as pltpu
from jax.experimental.pallas import tpu_sc as plsc

_NUM_CORES = 2
_NUM_SUBCORES = 16
_NUM_WORKERS = _NUM_CORES * _NUM_SUBCORES
_N_BUF = 4


@functools.partial(jax.jit, static_argnums=(2, 3, 4))
def _sc_gather(table, idx, B, D, C):
    b_per_w = B // _NUM_WORKERS
    n_chunks = b_per_w // C
    n_outer = n_chunks // _N_BUF
    mesh = plsc.VectorSubcoreMesh(core_axis_name="c", subcore_axis_name="s")

    @functools.partial(
        pl.kernel,
        mesh=mesh,
        out_type=jax.ShapeDtypeStruct((B, D), jnp.float32),
        scratch_types=[
            pltpu.VMEM((b_per_w,), jnp.int32),
            pltpu.VMEM((_N_BUF, C, D), jnp.float32),
            pltpu.SemaphoreType.DMA((_N_BUF,)),
        ],
        compiler_params=pltpu.CompilerParams(use_tc_tiling_on_sc=False),
    )
    def k(table_hbm, idx_hbm, out_hbm, idx_v, rows_v, gsem):
        wid = lax.axis_index("s") * _NUM_CORES + lax.axis_index("c")
        base = wid * b_per_w
        pltpu.sync_copy(idx_hbm.at[pl.ds(base, b_per_w)], idx_v)
        for b in range(_N_BUF):
            pltpu.async_copy(
                table_hbm.at[idx_v.at[pl.ds(b * C, C)]], rows_v.at[b], gsem.at[b]
            )

        def outer(g, carry):
            for b in range(_N_BUF):
                chunk = g * _N_BUF + b
                pltpu.make_async_copy(
                    table_hbm.at[idx_v.at[pl.ds(0, C)]], rows_v.at[b], gsem.at[b]
                ).wait()
                pltpu.sync_copy(
                    rows_v.at[b], out_hbm.at[pl.ds(base + chunk * C, C)]
                )

                @pl.when(g < n_outer - 1)
                def _():
                    pltpu.async_copy(
                        table_hbm.at[idx_v.at[pl.ds((chunk + _N_BUF) * C, C)]],
                        rows_v.at[b],
                        gsem.at[b],
                    )

            return carry

        lax.fori_loop(0, n_outer, outer, 0)

    return k(table, idx)


def kernel(table, x):
    batch, hist = x.shape
    D = table.shape[1]
    B = batch * hist
    idx = x.reshape(-1).astype(jnp.int32)
    out = _sc_gather(table, idx, B, D, 800)
    return out.reshape(batch, hist, D)


# speedup vs baseline: 1.4506x; 1.3030x over previous
import functools

import jax
import jax.numpy as jnp
from jax import lax
from jax.experimental import pallas as pl
from jax.experimental.pallas import tpu as pltpu
from jax.experimental.pallas import tpu_sc as plsc

_NUM_CORES = 2
_NUM_SUBCORES = 16
_NUM_WORKERS = _NUM_CORES * _NUM_SUBCORES
_L = 16


@functools.partial(jax.jit, static_argnums=(2, 3, 4))
def _sc_gather_t(table, idx, B, H, D):
    C = B // _NUM_WORKERS
    mesh = plsc.VectorSubcoreMesh(core_axis_name="c", subcore_axis_name="s")

    @functools.partial(
        pl.kernel,
        mesh=mesh,
        out_type=jax.ShapeDtypeStruct((H, D, B), jnp.float32),
        scratch_types=[
            pltpu.VMEM((C,), jnp.int32),
            pltpu.VMEM((C,), jnp.int32),
            pltpu.VMEM((C, D), jnp.float32),
            pltpu.VMEM((C, D), jnp.float32),
            pltpu.VMEM((D, C), jnp.float32),
            pltpu.VMEM((D, C), jnp.float32),
            pltpu.SemaphoreType.DMA,
            pltpu.SemaphoreType.DMA,
            pltpu.SemaphoreType.DMA,
            pltpu.SemaphoreType.DMA,
        ],
        compiler_params=pltpu.CompilerParams(
            use_tc_tiling_on_sc=False, needs_layout_passes=False
        ),
    )
    def k(table_hbm, idx_hbm, out_hbm, i0, i1, r0, r1, t0, t1, g0, g1, w0, w1):
        idxv, rows, trows = (i0, i1), (r0, r1), (t0, t1)
        gsem, wsem = (g0, g1), (w0, w1)
        wid = lax.axis_index("s") * _NUM_CORES + lax.axis_index("c")
        base = wid * C
        iota = lax.iota(jnp.int32, _L)
        dvecs = [jnp.full((_L,), d, jnp.int32) for d in range(D)]

        for b in range(2):
            pltpu.sync_copy(idx_hbm.at[pl.ds(b * B + base, C)], idxv[b])
            pltpu.async_copy(table_hbm.at[idxv[b]], rows[b], gsem[b])

        def outer(g, carry):
            for b in range(2):
                h = g * 2 + b
                pltpu.make_async_copy(
                    table_hbm.at[idxv[b]], rows[b], gsem[b]
                ).wait()

                @pl.when(g > 0)
                def _():
                    for d in range(D):
                        pltpu.make_async_copy(
                            trows[b].at[d],
                            out_hbm.at[h, d, pl.ds(base, C)],
                            wsem[b],
                        ).wait()

                def tbody(o, tc):
                    jv = o * _L + iota
                    for d in range(D):
                        v = plsc.load_gather(rows[b], [jv, dvecs[d]])
                        trows[b][d, pl.ds(o * _L, _L)] = v
                    return tc

                lax.fori_loop(0, C // _L, tbody, 0)

                for d in range(D):
                    pltpu.async_copy(
                        trows[b].at[d], out_hbm.at[h, d, pl.ds(base, C)], wsem[b]
                    )

                @pl.when(h + 2 < H)
                def _():
                    pltpu.sync_copy(
                        idx_hbm.at[pl.ds((h + 2) * B + base, C)], idxv[b]
                    )
                    pltpu.async_copy(table_hbm.at[idxv[b]], rows[b], gsem[b])

            return carry

        lax.fori_loop(0, H // 2, outer, 0)

        for b in range(2):
            for d in range(D):
                pltpu.make_async_copy(
                    trows[b].at[d], out_hbm.at[0, d, pl.ds(base, C)], wsem[b]
                ).wait()

    return k(table, idx)


def kernel(table, x):
    batch, hist = x.shape
    D = table.shape[1]
    idx = x.T.reshape(-1).astype(jnp.int32)
    out_t = _sc_gather_t(table, idx, batch, hist, D)
    return jnp.transpose(out_t, (2, 0, 1))


# speedup vs baseline: 1.8996x; 1.3095x over previous
import functools

import jax
import jax.numpy as jnp
from jax import lax
from jax.experimental import pallas as pl
from jax.experimental.pallas import tpu as pltpu
from jax.experimental.pallas import tpu_sc as plsc

_NUM_CORES = 2
_NUM_SUBCORES = 16
_NUM_WORKERS = _NUM_CORES * _NUM_SUBCORES
_L = 16
_UNROLL = 2


@functools.partial(jax.jit, static_argnums=(2, 3, 4))
def _sc_gather_t(table, idx, B, H, D):
    C = B // _NUM_WORKERS
    mesh = plsc.VectorSubcoreMesh(core_axis_name="c", subcore_axis_name="s")

    @functools.partial(
        pl.kernel,
        mesh=mesh,
        out_type=jax.ShapeDtypeStruct((H, D, B), jnp.float32),
        scratch_types=[
            pltpu.VMEM((H, C), jnp.int32),
            pltpu.VMEM((C, D), jnp.float32),
            pltpu.VMEM((C, D), jnp.float32),
            pltpu.VMEM((D, C), jnp.float32),
            pltpu.VMEM((D, C), jnp.float32),
            pltpu.SemaphoreType.DMA,
            pltpu.SemaphoreType.DMA,
            pltpu.SemaphoreType.DMA,
            pltpu.SemaphoreType.DMA,
        ],
        compiler_params=pltpu.CompilerParams(
            use_tc_tiling_on_sc=False, needs_layout_passes=False
        ),
    )
    def k(table_hbm, idx_hbm, out_hbm, idxv, r0, r1, t0, t1, g0, g1, w0, w1):
        rows, trows = (r0, r1), (t0, t1)
        gsem, wsem = (g0, g1), (w0, w1)
        wid = lax.axis_index("s") * _NUM_CORES + lax.axis_index("c")
        base = wid * C
        iota = lax.iota(jnp.int32, _L)
        dvecs = [jnp.full((_L,), d, jnp.int32) for d in range(D)]

        for h in range(H):
            pltpu.async_copy(
                idx_hbm.at[pl.ds(h * B + base, C)], idxv.at[h], g0
            )
        for h in range(H):
            pltpu.make_async_copy(
                idx_hbm.at[pl.ds(h * B + base, C)], idxv.at[h], g0
            ).wait()
        for b in range(2):
            pltpu.async_copy(table_hbm.at[idxv.at[b]], rows[b], gsem[b])

        def outer(g, carry):
            for b in range(2):
                h = g * 2 + b
                pltpu.make_async_copy(
                    table_hbm.at[idxv.at[b]], rows[b], gsem[b]
                ).wait()

                @pl.when(g > 0)
                def _():
                    pltpu.make_async_copy(
                        trows[b], out_hbm.at[h, :, pl.ds(base, C)], wsem[b]
                    ).wait()

                def tbody(o, tc):
                    for u in range(_UNROLL):
                        jv = (o * _UNROLL + u) * _L + iota
                        vals = [
                            plsc.load_gather(rows[b], [jv, dvecs[d]])
                            for d in range(D)
                        ]
                        for d in range(D):
                            trows[b][d, pl.ds((o * _UNROLL + u) * _L, _L)] = (
                                vals[d]
                            )
                    return tc

                lax.fori_loop(0, C // (_L * _UNROLL), tbody, 0)

                pltpu.async_copy(
                    trows[b], out_hbm.at[h, :, pl.ds(base, C)], wsem[b]
                )

                @pl.when(h + 2 < H)
                def _():
                    pltpu.async_copy(
                        table_hbm.at[idxv.at[h + 2]], rows[b], gsem[b]
                    )

            return carry

        lax.fori_loop(0, H // 2, outer, 0)

        for b in range(2):
            pltpu.make_async_copy(
                trows[b], out_hbm.at[0, :, pl.ds(base, C)], wsem[b]
            ).wait()

    return k(table, idx)


def kernel(table, x):
    batch, hist = x.shape
    D = table.shape[1]
    idx = x.T.reshape(-1).astype(jnp.int32)
    out_t = _sc_gather_t(table, idx, batch, hist, D)
    return jnp.transpose(out_t, (2, 0, 1))
